# baseline (speedup 1.0000x reference)
import functools

import jax
import jax.numpy as jnp
from jax import lax
from jax.experimental import pallas as pl
from jax.experimental.pallas import tpu as pltpu
from jax.experimental.pallas import tpu_sc as plsc

_MIN_DIST = 8.0
_STEP_DIST = 0.1
_NUM_BINS = 140

_B, _H, _W = 8, 256, 256
_N = _B * _H * _W
_NW = 32
_PER_W = _N // _NW
_C = 2048
_STEPS = _PER_W // _C
_L = 16

_BL = 64


def _sc_bin_indices(ipt_flat):
    mesh = plsc.VectorSubcoreMesh(core_axis_name="c", subcore_axis_name="s")

    @functools.partial(
        pl.kernel,
        mesh=mesh,
        out_type=jax.ShapeDtypeStruct((_N,), jnp.int32),
        compiler_params=pltpu.CompilerParams(
            needs_layout_passes=False, use_tc_tiling_on_sc=False),
        scratch_types=[
            pltpu.VMEM((_C,), jnp.float32),
            pltpu.VMEM((_C,), jnp.float32),
            pltpu.VMEM((_C,), jnp.int32),
            pltpu.VMEM((_C,), jnp.int32),
            pltpu.SemaphoreType.DMA,
            pltpu.SemaphoreType.DMA,
            pltpu.SemaphoreType.DMA,
            pltpu.SemaphoreType.DMA,
        ],
    )
    def k(ipt_hbm, idx_hbm, in0, in1, ob0, ob1, sin0, sin1, sout0, sout1):
        wid = lax.axis_index("s") * 2 + lax.axis_index("c")
        base = wid * _PER_W

        pltpu.async_copy(ipt_hbm.at[pl.ds(base, _C)], in0, sin0)
        pltpu.async_copy(ipt_hbm.at[pl.ds(base + _C, _C)], in1, sin1)

        bufs = ((in0, ob0, sin0, sout0), (in1, ob1, sin1, sout1))

        def step_pair(i, carry):
            for b in range(2):
                inb, ob, s_in, s_out = bufs[b]
                s = i * 2 + b

                @pl.when(s >= 2)
                def _wait_out():
                    pltpu.make_async_copy(
                        ob, idx_hbm.at[pl.ds(0, _C)], s_out).wait()

                pltpu.make_async_copy(
                    ipt_hbm.at[pl.ds(0, _C)], inb, s_in).wait()

                for j in range(_C // _L):
                    x = inb[pl.ds(j * _L, _L)]
                    v = ((x - _MIN_DIST) / _STEP_DIST).astype(jnp.int32)
                    v = jnp.minimum(jnp.maximum(v, 0), _NUM_BINS - 1)
                    ob[pl.ds(j * _L, _L)] = v

                pltpu.async_copy(
                    ob, idx_hbm.at[pl.ds(base + s * _C, _C)], s_out)

                @pl.when(s + 2 < _STEPS)
                def _prefetch():
                    pltpu.async_copy(
                        ipt_hbm.at[pl.ds(base + (s + 2) * _C, _C)], inb, s_in)
            return carry

        lax.fori_loop(0, _STEPS // 2, step_pair, 0)

        pltpu.make_async_copy(ob0, idx_hbm.at[pl.ds(0, _C)], sout0).wait()
        pltpu.make_async_copy(ob1, idx_hbm.at[pl.ds(0, _C)], sout1).wait()

    return k(ipt_flat)


def _tc_expand_body(idx_ref, o_ref):
    idx = idx_ref[...]
    k = jax.lax.broadcasted_iota(jnp.int32, (1, _BL, _W, _NUM_BINS), 3)
    o_ref[...] = (idx[..., None] == k).astype(jnp.float32)


def kernel(ipt, table):
    del table
    idx = _sc_bin_indices(ipt.reshape(-1)).reshape(_B, _H, _W)
    return pl.pallas_call(
        _tc_expand_body,
        grid=(_B, _H // _BL),
        in_specs=[pl.BlockSpec((1, _BL, _W), lambda b, h: (b, h, 0))],
        out_specs=pl.BlockSpec((1, _BL, _W, _NUM_BINS),
                               lambda b, h: (b, h, 0, 0)),
        out_shape=jax.ShapeDtypeStruct((_B, _H, _W, _NUM_BINS), jnp.float32),
    )(idx)


# speedup vs baseline: 1.0047x; 1.0047x over previous
import functools

import jax
import jax.numpy as jnp
from jax import lax
from jax.experimental import pallas as pl
from jax.experimental.pallas import tpu as pltpu
from jax.experimental.pallas import tpu_sc as plsc

_MIN_DIST = 8.0
_STEP_DIST = 0.1
_NUM_BINS = 140

_B, _H, _W = 8, 256, 256
_N = _B * _H * _W
_NW = 32
_PER_W = _N // _NW
_C = 2048
_STEPS = _PER_W // _C
_L = 16

_BL = 32


def _sc_bin_indices(ipt_flat):
    mesh = plsc.VectorSubcoreMesh(core_axis_name="c", subcore_axis_name="s")

    @functools.partial(
        pl.kernel,
        mesh=mesh,
        out_type=jax.ShapeDtypeStruct((_N,), jnp.int32),
        compiler_params=pltpu.CompilerParams(
            needs_layout_passes=False, use_tc_tiling_on_sc=False),
        scratch_types=[
            pltpu.VMEM((_C,), jnp.float32),
            pltpu.VMEM((_C,), jnp.float32),
            pltpu.VMEM((_C,), jnp.int32),
            pltpu.VMEM((_C,), jnp.int32),
            pltpu.SemaphoreType.DMA,
            pltpu.SemaphoreType.DMA,
            pltpu.SemaphoreType.DMA,
            pltpu.SemaphoreType.DMA,
        ],
    )
    def k(ipt_hbm, idx_hbm, in0, in1, ob0, ob1, sin0, sin1, sout0, sout1):
        wid = lax.axis_index("s") * 2 + lax.axis_index("c")
        base = wid * _PER_W

        pltpu.async_copy(ipt_hbm.at[pl.ds(base, _C)], in0, sin0)
        pltpu.async_copy(ipt_hbm.at[pl.ds(base + _C, _C)], in1, sin1)

        bufs = ((in0, ob0, sin0, sout0), (in1, ob1, sin1, sout1))

        def step_pair(i, carry):
            for b in range(2):
                inb, ob, s_in, s_out = bufs[b]
                s = i * 2 + b

                @pl.when(s >= 2)
                def _wait_out():
                    pltpu.make_async_copy(
                        ob, idx_hbm.at[pl.ds(0, _C)], s_out).wait()

                pltpu.make_async_copy(
                    ipt_hbm.at[pl.ds(0, _C)], inb, s_in).wait()

                for j in range(_C // _L):
                    x = inb[pl.ds(j * _L, _L)]
                    v = ((x - _MIN_DIST) / _STEP_DIST).astype(jnp.int32)
                    v = jnp.minimum(jnp.maximum(v, 0), _NUM_BINS - 1)
                    ob[pl.ds(j * _L, _L)] = v

                pltpu.async_copy(
                    ob, idx_hbm.at[pl.ds(base + s * _C, _C)], s_out)

                @pl.when(s + 2 < _STEPS)
                def _prefetch():
                    pltpu.async_copy(
                        ipt_hbm.at[pl.ds(base + (s + 2) * _C, _C)], inb, s_in)
            return carry

        lax.fori_loop(0, _STEPS // 2, step_pair, 0)

        pltpu.make_async_copy(ob0, idx_hbm.at[pl.ds(0, _C)], sout0).wait()
        pltpu.make_async_copy(ob1, idx_hbm.at[pl.ds(0, _C)], sout1).wait()

    return k(ipt_flat)


def _tc_expand_body(idx_ref, o_ref):
    idx = idx_ref[...]
    k = jax.lax.broadcasted_iota(jnp.int32, (1, _BL, _W, _NUM_BINS), 3)
    o_ref[...] = (idx[..., None] == k).astype(jnp.float32)


def kernel(ipt, table):
    del table
    idx = _sc_bin_indices(ipt.reshape(-1)).reshape(_B, _H, _W)
    return pl.pallas_call(
        _tc_expand_body,
        grid=(_B, _H // _BL),
        in_specs=[pl.BlockSpec((1, _BL, _W), lambda b, h: (b, h, 0))],
        out_specs=pl.BlockSpec((1, _BL, _W, _NUM_BINS),
                               lambda b, h: (b, h, 0, 0)),
        out_shape=jax.ShapeDtypeStruct((_B, _H, _W, _NUM_BINS), jnp.float32),
    )(idx)
